# baseline (speedup 1.0000x reference)
import functools

import jax
import jax.numpy as jnp
from jax.experimental import pallas as pl

P = 0.2
ROWS_PER_BLOCK = 16


def _scale_kernel(r_ref, xs_ref, scale_ref):
    r0 = r_ref[:, 0:1]
    r1 = r_ref[:, 1:2]
    xs = xs_ref[0:1, :]
    ch_index = jnp.sum((r0 > xs).astype(jnp.int32), axis=1, keepdims=True)
    active = (r1 < P).astype(jnp.float32)
    c_iota = jax.lax.broadcasted_iota(jnp.int32, scale_ref.shape, 1)
    onehot = (c_iota == ch_index).astype(jnp.float32)
    scale_ref[...] = 1.0 - onehot * active


def _mul_kernel(x_ref, s_ref, o_ref):
    o_ref[...] = x_ref[...] * s_ref[...]


def kernel(tensor, r):
    B, C, H, W = tensor.shape
    HW = H * W
    xs = jnp.linspace(1.0 / C, 1.0, C).reshape(1, C).astype(jnp.float32)

    scale = pl.pallas_call(
        _scale_kernel,
        out_shape=jax.ShapeDtypeStruct((B, C), jnp.float32),
    )(r, xs)

    x = tensor.reshape(B * C, HW)
    s = scale.reshape(B * C, 1)

    R = ROWS_PER_BLOCK
    grid = (B * C // R,)
    out = pl.pallas_call(
        _mul_kernel,
        grid=grid,
        in_specs=[
            pl.BlockSpec((R, HW), lambda i: (i, 0)),
            pl.BlockSpec((R, 1), lambda i: (i, 0)),
        ],
        out_specs=pl.BlockSpec((R, HW), lambda i: (i, 0)),
        out_shape=jax.ShapeDtypeStruct((B * C, HW), jnp.float32),
    )(x, s)
    return out.reshape(B, C, H, W)


# speedup vs baseline: 3.2449x; 3.2449x over previous
import jax
import jax.numpy as jnp
from jax.experimental import pallas as pl

P = 0.2
CH_PER_BLOCK = 8


def _scale_kernel(r_ref, xs_ref, scale_ref):
    r0 = r_ref[:, 0:1]
    r1 = r_ref[:, 1:2]
    xs = xs_ref[0:1, :]
    ch_index = jnp.sum((r0 > xs).astype(jnp.int32), axis=1, keepdims=True)
    active = (r1 < P).astype(jnp.float32)
    c_iota = jax.lax.broadcasted_iota(jnp.int32, scale_ref.shape, 1)
    onehot = (c_iota == ch_index).astype(jnp.float32)
    scale_ref[...] = 1.0 - onehot * active


def _mul_kernel(x_ref, s_ref, o_ref):
    o_ref[...] = x_ref[...] * s_ref[...]


def kernel(tensor, r):
    B, C, H, W = tensor.shape
    xs = jnp.linspace(1.0 / C, 1.0, C).reshape(1, C).astype(jnp.float32)

    scale = pl.pallas_call(
        _scale_kernel,
        out_shape=jax.ShapeDtypeStruct((B, C), jnp.float32),
    )(r, xs)
    s4 = scale.reshape(B, C, 1, 1)

    CB = CH_PER_BLOCK
    grid = (B, C // CB)
    out = pl.pallas_call(
        _mul_kernel,
        grid=grid,
        in_specs=[
            pl.BlockSpec((1, CB, H, W), lambda b, j: (b, j, 0, 0)),
            pl.BlockSpec((1, CB, 1, 1), lambda b, j: (b, j, 0, 0)),
        ],
        out_specs=pl.BlockSpec((1, CB, H, W), lambda b, j: (b, j, 0, 0)),
        out_shape=jax.ShapeDtypeStruct((B, C, H, W), jnp.float32),
    )(tensor, s4)
    return out


# speedup vs baseline: 3.2565x; 1.0036x over previous
import jax
import jax.numpy as jnp
from jax.experimental import pallas as pl
from jax.experimental.pallas import tpu as pltpu

P = 0.2
CH_PER_BLOCK = 8


def _drop_kernel(r_ref, xs_ref, cidx_ref, x_ref, o_ref):
    b = pl.program_id(0)
    r0 = r_ref[b, 0]
    r1 = r_ref[b, 1]
    ch_index = jnp.sum((r0 > xs_ref[...]).astype(jnp.int32))
    active = (r1 < P).astype(jnp.float32)
    onehot = (cidx_ref[...] == ch_index).astype(jnp.float32)
    scale = 1.0 - onehot * active
    o_ref[...] = x_ref[...] * scale


def kernel(tensor, r):
    B, C, H, W = tensor.shape
    xs = jnp.linspace(1.0 / C, 1.0, C).reshape(1, C).astype(jnp.float32)
    cidx = jnp.arange(C, dtype=jnp.int32).reshape(1, C, 1, 1)

    CB = CH_PER_BLOCK
    grid = (B, C // CB)
    out = pl.pallas_call(
        _drop_kernel,
        grid=grid,
        in_specs=[
            pl.BlockSpec(memory_space=pltpu.SMEM),
            pl.BlockSpec((1, C), lambda b, j: (0, 0)),
            pl.BlockSpec((1, CB, 1, 1), lambda b, j: (0, j, 0, 0)),
            pl.BlockSpec((1, CB, H, W), lambda b, j: (b, j, 0, 0)),
        ],
        out_specs=pl.BlockSpec((1, CB, H, W), lambda b, j: (b, j, 0, 0)),
        out_shape=jax.ShapeDtypeStruct((B, C, H, W), jnp.float32),
    )(r, xs, cidx, tensor)
    return out


# speedup vs baseline: 3.2757x; 1.0059x over previous
import jax
import jax.numpy as jnp
from jax.experimental import pallas as pl
from jax.experimental.pallas import tpu as pltpu

P = 0.2
CH_PER_BLOCK = 16


def _drop_kernel(r_ref, xs_ref, cidx_ref, x_ref, o_ref):
    b = pl.program_id(0)
    r0 = r_ref[b, 0]
    r1 = r_ref[b, 1]
    ch_index = jnp.sum((r0 > xs_ref[...]).astype(jnp.int32))
    active = (r1 < P).astype(jnp.float32)
    onehot = (cidx_ref[...] == ch_index).astype(jnp.float32)
    scale = 1.0 - onehot * active
    o_ref[...] = x_ref[...] * scale


def kernel(tensor, r):
    B, C, H, W = tensor.shape
    xs = jnp.linspace(1.0 / C, 1.0, C).reshape(1, C).astype(jnp.float32)
    cidx = jnp.arange(C, dtype=jnp.int32).reshape(1, C, 1, 1)

    CB = CH_PER_BLOCK
    grid = (B, C // CB)
    out = pl.pallas_call(
        _drop_kernel,
        grid=grid,
        in_specs=[
            pl.BlockSpec(memory_space=pltpu.SMEM),
            pl.BlockSpec((1, C), lambda b, j: (0, 0)),
            pl.BlockSpec((1, CB, 1, 1), lambda b, j: (0, j, 0, 0)),
            pl.BlockSpec((1, CB, H, W), lambda b, j: (b, j, 0, 0)),
        ],
        out_specs=pl.BlockSpec((1, CB, H, W), lambda b, j: (b, j, 0, 0)),
        out_shape=jax.ShapeDtypeStruct((B, C, H, W), jnp.float32),
    )(r, xs, cidx, tensor)
    return out


# speedup vs baseline: 3.2842x; 1.0026x over previous
import jax
import jax.numpy as jnp
from jax.experimental import pallas as pl
from jax.experimental.pallas import tpu as pltpu

P = 0.2
CH_PER_BLOCK = 24


def _drop_kernel(r_ref, xs_ref, cidx_ref, x_ref, o_ref):
    b = pl.program_id(0)
    r0 = r_ref[b, 0]
    r1 = r_ref[b, 1]
    ch_index = jnp.sum((r0 > xs_ref[...]).astype(jnp.int32))
    active = (r1 < P).astype(jnp.float32)
    onehot = (cidx_ref[...] == ch_index).astype(jnp.float32)
    scale = 1.0 - onehot * active
    o_ref[...] = x_ref[...] * scale


def kernel(tensor, r):
    B, C, H, W = tensor.shape
    xs = jnp.linspace(1.0 / C, 1.0, C).reshape(1, C).astype(jnp.float32)
    cidx = jnp.arange(C, dtype=jnp.int32).reshape(1, C, 1, 1)

    CB = CH_PER_BLOCK
    grid = (B, C // CB)
    out = pl.pallas_call(
        _drop_kernel,
        grid=grid,
        in_specs=[
            pl.BlockSpec(memory_space=pltpu.SMEM),
            pl.BlockSpec((1, C), lambda b, j: (0, 0)),
            pl.BlockSpec((1, CB, 1, 1), lambda b, j: (0, j, 0, 0)),
            pl.BlockSpec((1, CB, H, W), lambda b, j: (b, j, 0, 0)),
        ],
        out_specs=pl.BlockSpec((1, CB, H, W), lambda b, j: (b, j, 0, 0)),
        out_shape=jax.ShapeDtypeStruct((B, C, H, W), jnp.float32),
    )(r, xs, cidx, tensor)
    return out
